# baseline (speedup 1.0000x reference)
import jax
import jax.numpy as jnp
from jax.experimental import pallas as pl

N_PS = 50000
N_GW = 50000
N_SW = 50000
E = 150000
D = 128
OUT = 128

ROW_BLK = 1000


def _post_body(x_ps, x_gw, x_sw,
               s1, c1, s2, c2, s3, c3, s4, c4,
               wl1, wl2, wl3, wl4, wr1, wr24, wr3,
               bgw, bps, bsw, wlin, misc,
               out_ps, out_gwlin, out_sw):
    def mean(s_ref, c_ref):
        cnt = jnp.sum(c_ref[...], axis=1, keepdims=True)
        return s_ref[...] / jnp.maximum(cnt, 1.0)

    m1 = mean(s1, c1)
    h_gw = (jnp.dot(m1, wl1[...], preferred_element_type=jnp.float32)
            + jnp.dot(x_gw[...], wr1[...], preferred_element_type=jnp.float32)
            + bgw[...])
    r_gw = jnp.maximum(h_gw, 0.0)
    blin = misc[0, 0]
    a = misc[0, 1]
    g = jnp.sum(r_gw * wlin[...], axis=1, keepdims=True) + blin
    out_gwlin[...] = jnp.where(g >= 0, g, a * g)

    m2 = mean(s2, c2)
    m4 = mean(s4, c4)
    h_ps = (jnp.dot(m2, wl2[...], preferred_element_type=jnp.float32)
            + jnp.dot(m4, wl4[...], preferred_element_type=jnp.float32)
            + jnp.dot(x_ps[...], wr24[...], preferred_element_type=jnp.float32)
            + bps[...])
    out_ps[...] = jnp.maximum(h_ps, 0.0)

    m3 = mean(s3, c3)
    h_sw = (jnp.dot(m3, wl3[...], preferred_element_type=jnp.float32)
            + jnp.dot(x_sw[...], wr3[...], preferred_element_type=jnp.float32)
            + bsw[...])
    out_sw[...] = jnp.maximum(h_sw, 0.0)


def _post(x_ps, x_gw, x_sw, s1, c1, s2, c2, s3, c3, s4, c4,
          wl1, wl2, wl3, wl4, wr1, wr24, wr3, bgw, bps, bsw, wlin, misc):
    n = N_PS
    grid = (n // ROW_BLK,)
    row = pl.BlockSpec((ROW_BLK, D), lambda i: (i, 0))
    cnt = pl.BlockSpec((ROW_BLK, 16), lambda i: (i, 0))
    w = pl.BlockSpec((D, D), lambda i: (0, 0))
    b = pl.BlockSpec((1, D), lambda i: (0, 0))
    return pl.pallas_call(
        _post_body,
        grid=grid,
        in_specs=[row, row, row,
                  row, cnt, row, cnt, row, cnt, row, cnt,
                  w, w, w, w, w, w, w,
                  b, b, b, b, b],
        out_specs=[row,
                   pl.BlockSpec((ROW_BLK, 1), lambda i: (i, 0)),
                   row],
        out_shape=[jax.ShapeDtypeStruct((n, OUT), jnp.float32),
                   jax.ShapeDtypeStruct((n, 1), jnp.float32),
                   jax.ShapeDtypeStruct((n, OUT), jnp.float32)],
    )(x_ps, x_gw, x_sw, s1, c1, s2, c2, s3, c3, s4, c4,
      wl1, wl2, wl3, wl4, wr1, wr24, wr3, bgw, bps, bsw, wlin, misc)


def _seg(x_src, src, dst, n_dst):
    msgs = jnp.take(x_src, src, axis=0)
    summed = jax.ops.segment_sum(msgs, dst, num_segments=n_dst)
    cnt = jax.ops.segment_sum(jnp.ones_like(dst, dtype=jnp.float32), dst,
                              num_segments=n_dst)
    return summed, cnt


def kernel(x_pfas_sites, x_gw_wells, x_sw_stations, ei_ps_gw, ei_gw_ps,
           ei_ps_sw, ei_sw_ps, Wl1, bl1, Wr1, Wl2, bl2, Wr2, Wl3, bl3, Wr3,
           Wl4, bl4, Wr4, Wlin, blin, prelu_a):
    s1, c1 = _seg(x_pfas_sites, ei_ps_gw[0], ei_ps_gw[1], N_GW)
    s2, c2 = _seg(x_gw_wells, ei_gw_ps[0], ei_gw_ps[1], N_PS)
    s3, c3 = _seg(x_pfas_sites, ei_ps_sw[0], ei_ps_sw[1], N_SW)
    s4, c4 = _seg(x_sw_stations, ei_sw_ps[0], ei_sw_ps[1], N_PS)

    def c16(c):
        z = jnp.zeros((c.shape[0], 16), jnp.float32)
        return z.at[:, 0].set(c)

    misc = jnp.stack([blin[0], prelu_a]).reshape(1, 2)
    misc = jnp.pad(misc, ((0, 0), (0, D - 2)))
    out_ps, gw, out_sw = _post(
        x_pfas_sites, x_gw_wells, x_sw_stations,
        s1, c16(c1), s2, c16(c2), s3, c16(c3), s4, c16(c4),
        Wl1.T, Wl2.T, Wl3.T, Wl4.T, Wr1.T, (Wr2 + Wr4).T, Wr3.T,
        bl1.reshape(1, D), (bl2 + bl4).reshape(1, D), bl3.reshape(1, D),
        Wlin.reshape(1, OUT), misc)
    return (out_ps, gw, out_sw)
